# baseline (speedup 1.0000x reference)
import functools
import math

import jax
import jax.numpy as jnp
from jax import lax
from jax.experimental import pallas as pl
from jax.experimental.pallas import tpu as pltpu
from jax.experimental.pallas import tpu_sc as plsc

_SIZE = 32000
_SMOOTHING = 0.1
_CONF = 1.0 - _SMOOTHING
_EPS = _SMOOTHING / (_SIZE - 2)
_N = 2048
_CBLK = 3200

_NC = 2
_NS = 16
_NW = _NC * _NS
_LANES = 16

_R0 = 1536
_RPW = (_N - _R0) // _NW
_SLAB = 640
_NSLAB = _SIZE // _SLAB
_APW = _N // _NW

_A_ZERO = (_SIZE - 1) * _EPS * math.log(_EPS)
_A_NONZERO = (_SIZE - 2) * _EPS * math.log(_EPS) + _CONF * math.log(_CONF)


def _dense_kernel(tgt_ref, pred_ref, out_ref):
    k = pl.program_id(0)
    eps = jnp.float32(_EPS)
    conf = jnp.float32(_CONF)

    t_all = tgt_ref[...]
    any_z = jnp.any(t_all == 0)
    any_nz = jnp.any(t_all != 0)

    t = t_all[:_R0]
    rid = jax.lax.broadcasted_iota(jnp.int32, (_R0, 1), 0)
    w = jnp.where((rid == 0) & any_nz, 0.0, 1.0) * jnp.where(
        (rid == 1) & any_z, 0.0, 1.0
    )

    block = pred_ref[...]
    col = jax.lax.broadcasted_iota(jnp.int32, (_R0, _CBLK), 1) + k * _CBLK
    sel = jnp.where(col == t, block, 0.0)
    rowsum = jnp.sum(block, axis=1, keepdims=True)
    psel = jnp.sum(sel, axis=1, keepdims=True)

    beta = jnp.where(t == 0, 0.0, eps - conf)
    partial = jnp.sum(w * (beta * psel - eps * rowsum))

    @pl.when(k == 0)
    def _init():
        p0 = block[:, 0:1]
        out_ref[...] = jnp.sum(w * eps * p0).reshape(1, 1)

    out_ref[...] += partial.reshape(1, 1)


def _sc_kernel(pred, tgt, out, tgt_v, buf0, buf1, p0_v, acc_v, sem0, sem1):
    wid = lax.axis_index("s") * _NC + lax.axis_index("c")
    r0 = _R0 + wid * _RPW
    eps = jnp.float32(_EPS)
    conf = jnp.float32(_CONF)
    lane = lax.iota(jnp.int32, _LANES)

    pltpu.sync_copy(tgt, tgt_v)

    bufs = (buf0, buf1)
    sems = (sem0, sem1)

    def _slab_copy(s, b):
        off = pl.multiple_of(s * _SLAB, _SLAB)
        return pltpu.make_async_copy(
            pred.at[pl.ds(r0, _RPW), pl.ds(off, _SLAB)], bufs[b], sems[b]
        )

    _slab_copy(jnp.int32(0), 0).start()
    _slab_copy(jnp.int32(1), 1).start()

    t_mine = tgt_v[pl.ds(r0, _LANES)]
    beta = jnp.where(t_mine == 0, jnp.float32(0.0), eps - conf)

    def _round(s2, carry):
        a0, a1, a2, a3, extra = carry
        for b in range(2):
            s = 2 * s2 + b
            _slab_copy(s, b).wait()

            in_slab = (t_mine >= s * _SLAB) & (t_mine < (s + 1) * _SLAB)
            idxj = jnp.clip(t_mine - s * _SLAB, 0, _SLAB - 1)
            pt = plsc.load_gather(bufs[b], [lane, idxj])
            extra = extra + jnp.where(in_slab, beta * pt, jnp.float32(0.0))

            def _sum_row(i, accs, b=b):
                def _sum4(j, accs4):
                    c0, c1, c2, c3 = accs4
                    base = j * 4 * _LANES
                    return (
                        c0 + bufs[b][i, pl.ds(base, _LANES)],
                        c1 + bufs[b][i, pl.ds(base + _LANES, _LANES)],
                        c2 + bufs[b][i, pl.ds(base + 2 * _LANES, _LANES)],
                        c3 + bufs[b][i, pl.ds(base + 3 * _LANES, _LANES)],
                    )

                return lax.fori_loop(
                    0, _SLAB // (4 * _LANES), _sum4, accs, unroll=4
                )

            a0, a1, a2, a3 = lax.fori_loop(
                0, _RPW, _sum_row, (a0, a1, a2, a3)
            )

            @pl.when(s + 2 < _NSLAB)
            def _start_next(s=s, b=b):
                _slab_copy(s + 2, b).start()

        return a0, a1, a2, a3, extra

    z16 = jnp.zeros((_LANES,), jnp.float32)
    a0, a1, a2, a3, extra = lax.fori_loop(
        0, _NSLAB // 2, _round, (z16, z16, z16, z16, z16)
    )
    total = (a0 + a1) + (a2 + a3)

    pltpu.sync_copy(pred.at[pl.ds(r0, _RPW), pl.ds(0, 128)], p0_v)
    p0 = plsc.load_gather(p0_v, [lane, jnp.zeros((_LANES,), jnp.int32)])
    extra = extra + eps * p0

    def _count(i, nz):
        chunk = tgt_v[pl.ds(i * _LANES, _LANES)]
        return nz + plsc.all_reduce_population_count(chunk == 0)

    num_zero = lax.fori_loop(
        0, _N // _LANES, _count, jnp.zeros((_LANES,), jnp.int32)
    )
    zero_row0 = jnp.where(num_zero < _N, jnp.float32(1.0), jnp.float32(0.0))
    zero_row1 = jnp.where(num_zero > 0, jnp.float32(1.0), jnp.float32(0.0))

    a_acc = jnp.zeros((_LANES,), jnp.float32)
    for j in range(_APW // _LANES):
        rows = wid * _APW + j * _LANES + lane
        t_chunk = tgt_v[pl.ds(wid * _APW + j * _LANES, _LANES)]
        a_i = jnp.where(
            t_chunk == 0, jnp.float32(_A_ZERO), jnp.float32(_A_NONZERO)
        )
        w = (
            jnp.float32(1.0)
            - jnp.where(rows == 0, zero_row0, jnp.float32(0.0))
            - jnp.where(rows == 1, zero_row1, jnp.float32(0.0))
        )
        a_acc = a_acc + w * a_i

    acc_v[...] = a_acc + extra - eps * total
    pltpu.sync_copy(acc_v, out.at[wid])


@functools.partial(jax.jit, static_argnames=("interpret",))
def kernel(prediction, target, interpret=False):
    n, size = prediction.shape
    tgt_i32 = target.astype(jnp.int32)

    sc_fn = pl.kernel(
        _sc_kernel,
        mesh=plsc.VectorSubcoreMesh(core_axis_name="c", subcore_axis_name="s"),
        out_type=jax.ShapeDtypeStruct((_NW, _LANES), jnp.float32),
        compiler_params=pltpu.CompilerParams(needs_layout_passes=False),
        scratch_types=[
            pltpu.VMEM((_N,), jnp.int32),
            pltpu.VMEM((_RPW, _SLAB), jnp.float32),
            pltpu.VMEM((_RPW, _SLAB), jnp.float32),
            pltpu.VMEM((_RPW, 128), jnp.float32),
            pltpu.VMEM((_LANES,), jnp.float32),
            pltpu.SemaphoreType.DMA,
            pltpu.SemaphoreType.DMA,
        ],
    )
    sc_part = sc_fn(prediction, tgt_i32)

    dense = pl.pallas_call(
        _dense_kernel,
        grid=(size // _CBLK,),
        in_specs=[
            pl.BlockSpec((n, 1), lambda k: (0, 0)),
            pl.BlockSpec((_R0, _CBLK), lambda k: (0, k)),
        ],
        out_specs=pl.BlockSpec((1, 1), lambda k: (0, 0)),
        out_shape=jax.ShapeDtypeStruct((1, 1), jnp.float32),
        interpret=interpret,
    )(tgt_i32.reshape(n, 1), prediction)

    return dense[0, 0] + jnp.sum(sc_part)


# speedup vs baseline: 1.0004x; 1.0004x over previous
import math

import jax
import jax.numpy as jnp
from jax import lax
from jax.experimental import pallas as pl
from jax.experimental.pallas import tpu as pltpu
from jax.experimental.pallas import tpu_sc as plsc

_SIZE = 32000
_SMOOTHING = 0.1
_CONF = 1.0 - _SMOOTHING
_EPS = _SMOOTHING / (_SIZE - 2)
_N = 2048
_CBLK = 3200

_NC = 2
_NS = 16
_NW = _NC * _NS
_LANES = 16

_R0 = 1536
_RPW = (_N - _R0) // _NW
_SLAB = 640
_NSLAB = _SIZE // _SLAB
_APW = _N // _NW

_A_ZERO = (_SIZE - 1) * _EPS * math.log(_EPS)
_A_NONZERO = (_SIZE - 2) * _EPS * math.log(_EPS) + _CONF * math.log(_CONF)


def _dense_kernel(tgt_ref, pred_ref, out_ref):
    k = pl.program_id(0)
    eps = jnp.float32(_EPS)
    conf = jnp.float32(_CONF)

    t_all = tgt_ref[...]
    any_z = jnp.any(t_all == 0)
    any_nz = jnp.any(t_all != 0)

    t = t_all[:_R0]
    rid = jax.lax.broadcasted_iota(jnp.int32, (_R0, 1), 0)
    w = jnp.where((rid == 0) & any_nz, 0.0, 1.0) * jnp.where(
        (rid == 1) & any_z, 0.0, 1.0
    )

    block = pred_ref[...]
    col = jax.lax.broadcasted_iota(jnp.int32, (_R0, _CBLK), 1) + k * _CBLK
    sel = jnp.where(col == t, block, 0.0)
    rowsum = jnp.sum(block, axis=1, keepdims=True)
    psel = jnp.sum(sel, axis=1, keepdims=True)

    beta = jnp.where(t == 0, 0.0, eps - conf)
    partial = jnp.sum(w * (beta * psel - eps * rowsum))

    @pl.when(k == 0)
    def _init():
        p0 = block[:, 0:1]
        out_ref[...] = jnp.sum(w * eps * p0).reshape(1, 1)

    out_ref[...] += partial.reshape(1, 1)


def _sc_kernel(pred, tgt, out, tgt_v, buf0, buf1, p0_v, acc_v, sem0, sem1):
    wid = lax.axis_index("s") * _NC + lax.axis_index("c")
    r0 = _R0 + wid * _RPW
    eps = jnp.float32(_EPS)
    conf = jnp.float32(_CONF)
    lane = lax.iota(jnp.int32, _LANES)

    pltpu.sync_copy(tgt, tgt_v)

    bufs = (buf0, buf1)
    sems = (sem0, sem1)

    def _slab_copy(s, b):
        off = pl.multiple_of(s * _SLAB, _SLAB)
        return pltpu.make_async_copy(
            pred.at[pl.ds(r0, _RPW), pl.ds(off, _SLAB)], bufs[b], sems[b]
        )

    _slab_copy(jnp.int32(0), 0).start()
    _slab_copy(jnp.int32(1), 1).start()

    t_mine = tgt_v[pl.ds(r0, _LANES)]
    beta = jnp.where(t_mine == 0, jnp.float32(0.0), eps - conf)

    def _round(s2, carry):
        a0, a1, a2, a3, extra = carry
        for b in range(2):
            s = 2 * s2 + b
            _slab_copy(s, b).wait()

            in_slab = (t_mine >= s * _SLAB) & (t_mine < (s + 1) * _SLAB)
            idxj = jnp.clip(t_mine - s * _SLAB, 0, _SLAB - 1)
            pt = plsc.load_gather(bufs[b], [lane, idxj])
            extra = extra + jnp.where(in_slab, beta * pt, jnp.float32(0.0))

            def _sum_row(i, accs, b=b):
                def _sum4(j, accs4):
                    c0, c1, c2, c3 = accs4
                    base = j * 4 * _LANES
                    return (
                        c0 + bufs[b][i, pl.ds(base, _LANES)],
                        c1 + bufs[b][i, pl.ds(base + _LANES, _LANES)],
                        c2 + bufs[b][i, pl.ds(base + 2 * _LANES, _LANES)],
                        c3 + bufs[b][i, pl.ds(base + 3 * _LANES, _LANES)],
                    )

                return lax.fori_loop(
                    0, _SLAB // (4 * _LANES), _sum4, accs, unroll=4
                )

            a0, a1, a2, a3 = lax.fori_loop(
                0, _RPW, _sum_row, (a0, a1, a2, a3)
            )

            @pl.when(s + 2 < _NSLAB)
            def _start_next(s=s, b=b):
                _slab_copy(s + 2, b).start()

        return a0, a1, a2, a3, extra

    z16 = jnp.zeros((_LANES,), jnp.float32)
    a0, a1, a2, a3, extra = lax.fori_loop(
        0, _NSLAB // 2, _round, (z16, z16, z16, z16, z16)
    )
    total = (a0 + a1) + (a2 + a3)

    pltpu.sync_copy(pred.at[pl.ds(r0, _RPW), pl.ds(0, 128)], p0_v)
    p0 = plsc.load_gather(p0_v, [lane, jnp.zeros((_LANES,), jnp.int32)])
    extra = extra + eps * p0

    def _count(i, nz):
        chunk = tgt_v[pl.ds(i * _LANES, _LANES)]
        return nz + plsc.all_reduce_population_count(chunk == 0)

    num_zero = lax.fori_loop(
        0, _N // _LANES, _count, jnp.zeros((_LANES,), jnp.int32)
    )
    zero_row0 = jnp.where(num_zero < _N, jnp.float32(1.0), jnp.float32(0.0))
    zero_row1 = jnp.where(num_zero > 0, jnp.float32(1.0), jnp.float32(0.0))

    a_acc = jnp.zeros((_LANES,), jnp.float32)
    for j in range(_APW // _LANES):
        rows = wid * _APW + j * _LANES + lane
        t_chunk = tgt_v[pl.ds(wid * _APW + j * _LANES, _LANES)]
        a_i = jnp.where(
            t_chunk == 0, jnp.float32(_A_ZERO), jnp.float32(_A_NONZERO)
        )
        w = (
            jnp.float32(1.0)
            - jnp.where(rows == 0, zero_row0, jnp.float32(0.0))
            - jnp.where(rows == 1, zero_row1, jnp.float32(0.0))
        )
        a_acc = a_acc + w * a_i

    acc_v[...] = a_acc + extra - eps * total
    pltpu.sync_copy(acc_v, out.at[wid])


@jax.jit
def kernel(prediction, target):
    n, size = prediction.shape
    tgt_i32 = target.astype(jnp.int32)

    sc_fn = pl.kernel(
        _sc_kernel,
        mesh=plsc.VectorSubcoreMesh(core_axis_name="c", subcore_axis_name="s"),
        out_type=jax.ShapeDtypeStruct((_NW, _LANES), jnp.float32),
        compiler_params=pltpu.CompilerParams(needs_layout_passes=False),
        scratch_types=[
            pltpu.VMEM((_N,), jnp.int32),
            pltpu.VMEM((_RPW, _SLAB), jnp.float32),
            pltpu.VMEM((_RPW, _SLAB), jnp.float32),
            pltpu.VMEM((_RPW, 128), jnp.float32),
            pltpu.VMEM((_LANES,), jnp.float32),
            pltpu.SemaphoreType.DMA,
            pltpu.SemaphoreType.DMA,
        ],
    )
    sc_part = sc_fn(prediction, tgt_i32)

    dense = pl.pallas_call(
        _dense_kernel,
        grid=(size // _CBLK,),
        in_specs=[
            pl.BlockSpec((n, 1), lambda k: (0, 0)),
            pl.BlockSpec((_R0, _CBLK), lambda k: (0, k)),
        ],
        out_specs=pl.BlockSpec((1, 1), lambda k: (0, 0)),
        out_shape=jax.ShapeDtypeStruct((1, 1), jnp.float32),
    )(tgt_i32.reshape(n, 1), prediction)

    return dense[0, 0] + jnp.sum(sc_part)
